# baseline (speedup 1.0000x reference)
import functools

import jax
import jax.numpy as jnp
from jax import lax
from jax.experimental import pallas as pl
from jax.experimental.pallas import tpu as pltpu
from jax.experimental.pallas import tpu_sc as plsc

N_NODES = 10000
N_EDGES = 320000
NC = 2
NS = 16
NW = NC * NS
B_E = 64
K_CH = 158
K2 = 314
E_W = K_CH * B_E
N_ACC = 10112
RPT = N_ACC // NS
DEG_W = 8


def _sc_conv_call(table, row3, col3, zeros, stage_table=False):
    D = table.shape[1]
    mesh = plsc.VectorSubcoreMesh(
        core_axis_name="c", subcore_axis_name="s", num_cores=NC, num_subcores=NS
    )
    scratch = [
        pltpu.VMEM((K_CH, B_E), jnp.int32),
        pltpu.VMEM((K_CH, B_E), jnp.int32),
        pltpu.VMEM((B_E, D), jnp.float32),
        pltpu.VMEM((B_E, D), jnp.float32),
        pltpu.SemaphoreType.DMA,
        pltpu.SemaphoreType.DMA,
        pltpu.VMEM_SHARED((N_ACC, D), jnp.float32),
    ]
    if stage_table:
        scratch.append(pltpu.VMEM_SHARED((N_ACC, D), jnp.float32))

    @functools.partial(
        pl.kernel,
        out_type=jax.ShapeDtypeStruct((NC, N_ACC, D), jnp.float32),
        mesh=mesh,
        scratch_types=scratch,
        compiler_params=pltpu.CompilerParams(use_tc_tiling_on_sc=False),
    )
    def conv(table_h, row_h, col_h, zeros_h, out_h, rowv, colv, buf0, buf1,
             sem0, sem1, acc, *maybe_tab):
        c = lax.axis_index("c")
        s = lax.axis_index("s")
        w = c * NS + s
        pltpu.sync_copy(zeros_h.at[pl.ds(s * RPT, RPT)], acc.at[pl.ds(s * RPT, RPT)])
        if stage_table:
            table_src = maybe_tab[0]
            pltpu.sync_copy(
                table_h.at[pl.ds(s * RPT, RPT)], table_src.at[pl.ds(s * RPT, RPT)]
            )
        else:
            table_src = table_h
        pltpu.sync_copy(row_h.at[w], rowv)
        pltpu.sync_copy(col_h.at[w], colv)
        plsc.subcore_barrier()

        pltpu.async_copy(table_src.at[rowv.at[0]], buf0, sem0)

        def body(jj, carry):
            j0 = 2 * jj
            pltpu.async_copy(table_src.at[rowv.at[j0 + 1]], buf1, sem1)
            pltpu.make_async_copy(table_src.at[rowv.at[j0]], buf0, sem0).wait()
            pltpu.sync_copy(buf0, acc.at[colv.at[j0]], add=True)
            pltpu.async_copy(table_src.at[rowv.at[j0 + 2]], buf0, sem0)
            pltpu.make_async_copy(table_src.at[rowv.at[j0 + 1]], buf1, sem1).wait()
            pltpu.sync_copy(buf1, acc.at[colv.at[j0 + 1]], add=True)
            return carry

        lax.fori_loop(0, (K_CH - 2) // 2, body, 0)
        pltpu.async_copy(table_src.at[rowv.at[K_CH - 1]], buf1, sem1)
        pltpu.make_async_copy(table_src.at[rowv.at[K_CH - 2]], buf0, sem0).wait()
        pltpu.sync_copy(buf0, acc.at[colv.at[K_CH - 2]], add=True)
        pltpu.make_async_copy(table_src.at[rowv.at[K_CH - 1]], buf1, sem1).wait()
        pltpu.sync_copy(buf1, acc.at[colv.at[K_CH - 1]], add=True)

        plsc.subcore_barrier()
        pltpu.sync_copy(
            acc.at[pl.ds(s * RPT, RPT)], out_h.at[c, pl.ds(s * RPT, RPT)]
        )

    return conv(table, row3, col3, zeros)


def _sc_conv128_call(table, row2, col2, zeros):
    DH = 64
    mesh = plsc.VectorSubcoreMesh(
        core_axis_name="c", subcore_axis_name="s", num_cores=NC, num_subcores=NS
    )

    @functools.partial(
        pl.kernel,
        out_type=jax.ShapeDtypeStruct((2, N_ACC, DH), jnp.float32),
        mesh=mesh,
        scratch_types=[
            pltpu.VMEM((K2, B_E), jnp.int32),
            pltpu.VMEM((K2, B_E), jnp.int32),
            pltpu.VMEM((B_E, DH), jnp.float32),
            pltpu.VMEM((B_E, DH), jnp.float32),
            pltpu.SemaphoreType.DMA,
            pltpu.SemaphoreType.DMA,
            pltpu.VMEM_SHARED((N_ACC, DH), jnp.float32),
            pltpu.VMEM_SHARED((N_ACC, DH), jnp.float32),
        ],
        compiler_params=pltpu.CompilerParams(use_tc_tiling_on_sc=False),
    )
    def conv(table_h, row_h, col_h, zeros_h, out_h, rowv, colv, buf0, buf1,
             sem0, sem1, acc, tab):
        c = lax.axis_index("c")
        s = lax.axis_index("s")
        pltpu.sync_copy(row_h.at[s], rowv)
        pltpu.sync_copy(col_h.at[s], colv)
        pltpu.sync_copy(zeros_h.at[pl.ds(s * RPT, RPT)], acc.at[pl.ds(s * RPT, RPT)])
        pltpu.sync_copy(
            table_h.at[pl.ds(s * RPT, RPT), pl.ds(c * DH, DH)],
            tab.at[pl.ds(s * RPT, RPT)],
        )
        plsc.subcore_barrier()

        pltpu.async_copy(tab.at[rowv.at[0]], buf0, sem0)

        def body(jj, carry):
            j0 = 2 * jj
            pltpu.async_copy(tab.at[rowv.at[j0 + 1]], buf1, sem1)
            pltpu.make_async_copy(tab.at[rowv.at[j0]], buf0, sem0).wait()
            pltpu.sync_copy(buf0, acc.at[colv.at[j0]], add=True)
            pltpu.async_copy(tab.at[rowv.at[j0 + 2]], buf0, sem0)
            pltpu.make_async_copy(tab.at[rowv.at[j0 + 1]], buf1, sem1).wait()
            pltpu.sync_copy(buf1, acc.at[colv.at[j0 + 1]], add=True)
            return carry

        lax.fori_loop(0, (K2 - 2) // 2, body, 0)
        pltpu.async_copy(tab.at[rowv.at[K2 - 1]], buf1, sem1)
        pltpu.make_async_copy(tab.at[rowv.at[K2 - 2]], buf0, sem0).wait()
        pltpu.sync_copy(buf0, acc.at[colv.at[K2 - 2]], add=True)
        pltpu.make_async_copy(tab.at[rowv.at[K2 - 1]], buf1, sem1).wait()
        pltpu.sync_copy(buf1, acc.at[colv.at[K2 - 1]], add=True)

        plsc.subcore_barrier()
        pltpu.sync_copy(
            acc.at[pl.ds(s * RPT, RPT)], out_h.at[c, pl.ds(s * RPT, RPT)]
        )

    return conv(table, row2, col2, zeros)


B_D = 128
K_D = K_CH // 2


def _sc_deg_call(ones_blk, col3d, zeros):
    mesh = plsc.VectorSubcoreMesh(
        core_axis_name="c", subcore_axis_name="s", num_cores=NC, num_subcores=NS
    )

    @functools.partial(
        pl.kernel,
        out_type=jax.ShapeDtypeStruct((NC, N_ACC, DEG_W), jnp.float32),
        mesh=mesh,
        scratch_types=[
            pltpu.VMEM((K_D, B_D), jnp.int32),
            pltpu.VMEM((B_D, DEG_W), jnp.float32),
            pltpu.VMEM_SHARED((N_ACC, DEG_W), jnp.float32),
        ],
        compiler_params=pltpu.CompilerParams(use_tc_tiling_on_sc=False),
    )
    def degk(ones_h, col_h, zeros_h, out_h, colv, onesv, acc):
        c = lax.axis_index("c")
        s = lax.axis_index("s")
        w = c * NS + s
        pltpu.sync_copy(zeros_h.at[pl.ds(s * RPT, RPT)], acc.at[pl.ds(s * RPT, RPT)])
        pltpu.sync_copy(col_h.at[w], colv)
        pltpu.sync_copy(ones_h, onesv)
        plsc.subcore_barrier()

        def body(j, carry):
            pltpu.sync_copy(onesv, acc.at[colv.at[j]], add=True)
            return carry

        lax.fori_loop(0, K_D, body, 0)
        plsc.subcore_barrier()
        pltpu.sync_copy(
            acc.at[pl.ds(s * RPT, RPT)], out_h.at[c, pl.ds(s * RPT, RPT)]
        )

    return degk(ones_blk, col3d, zeros)


def _dis_from(d_ref):
    deg = d_ref[0, :, :, 0:1].sum(axis=0)
    return jnp.where(deg > 0.0, lax.rsqrt(deg), 0.0)


def _mmscale_body(x_ref, w_ref, d_ref, o_ref):
    o_ref[...] = jnp.dot(
        x_ref[...] * _dis_from(d_ref), w_ref[...],
        preferred_element_type=jnp.float32,
    )


def _mm2_body(pa_ref, pb_ref, d_ref, wa_ref, wb_ref, o_ref):
    dis = _dis_from(d_ref)
    out1a = pa_ref[0] * dis
    out1b = pb_ref[0] * dis
    h2 = jnp.dot(out1a, wa_ref[...], preferred_element_type=jnp.float32)
    h2 += jnp.dot(out1b, wb_ref[...], preferred_element_type=jnp.float32)
    o_ref[...] = h2 * dis


def _fin_body(q_ref, d_ref, o_ref):
    dis = _dis_from(d_ref)
    a = ((q_ref[0] + q_ref[1]) * dis)[:, :40]
    m = jnp.max(a, axis=1, keepdims=True)
    e = jnp.exp(a - m)
    lse = jnp.log(jnp.sum(e, axis=1, keepdims=True))
    o_ref[...] = a - m - lse


_R = 632
_G = N_ACC // _R


@jax.jit
def kernel(x, edge_index, W1, W2):
    f32 = jnp.float32
    row = edge_index[0]
    col = edge_index[1]
    pad = NW * E_W - N_EDGES
    row_p = jnp.concatenate([row, jnp.zeros((pad,), jnp.int32)])
    col_p = jnp.concatenate([col, jnp.full((pad,), N_NODES, jnp.int32)])
    row3 = row_p.reshape(NW, K_CH, B_E)
    col3 = col_p.reshape(NW, K_CH, B_E)
    pad2 = NS * K2 * B_E - N_EDGES
    row2 = jnp.concatenate([row, jnp.zeros((pad2,), jnp.int32)]).reshape(NS, K2, B_E)
    col2 = jnp.concatenate([col, jnp.full((pad2,), N_NODES, jnp.int32)]).reshape(
        NS, K2, B_E
    )

    zeros64 = jnp.zeros((N_ACC, 64), f32)
    zeros48 = jnp.zeros((N_ACC, 48), f32)
    zeros16 = jnp.zeros((N_ACC, DEG_W), f32)
    ones_blk = jnp.ones((B_D, DEG_W), f32)

    deg_parts = _sc_deg_call(
        ones_blk, col3.reshape(NW, K_D, B_D), zeros16
    )
    h1p = pl.pallas_call(
        _mmscale_body,
        grid=(_G,),
        in_specs=[
            pl.BlockSpec((_R, 128), lambda i: (i, 0)),
            pl.BlockSpec((128, 128), lambda i: (0, 0)),
            pl.BlockSpec((1, 2, _R, DEG_W), lambda i: (0, 0, i, 0)),
        ],
        out_specs=pl.BlockSpec((_R, 128), lambda i: (i, 0)),
        out_shape=jax.ShapeDtypeStruct((N_ACC, 128), f32),
    )(x, W1, deg_parts[None])

    phalves = _sc_conv128_call(h1p, row2, col2, zeros64)

    W2p = jnp.pad(W2, ((0, 0), (0, 8)))
    h2p = pl.pallas_call(
        _mm2_body,
        grid=(_G,),
        in_specs=[
            pl.BlockSpec((1, _R, 64), lambda i: (0, i, 0)),
            pl.BlockSpec((1, _R, 64), lambda i: (1, i, 0)),
            pl.BlockSpec((1, 2, _R, DEG_W), lambda i: (0, 0, i, 0)),
            pl.BlockSpec((64, 48), lambda i: (0, 0)),
            pl.BlockSpec((64, 48), lambda i: (0, 0)),
        ],
        out_specs=pl.BlockSpec((_R, 48), lambda i: (i, 0)),
        out_shape=jax.ShapeDtypeStruct((N_ACC, 48), f32),
    )(phalves, phalves, deg_parts[None], W2p[:64], W2p[64:])

    q = _sc_conv_call(h2p, row3, col3, zeros48, stage_table=True)

    out = pl.pallas_call(
        _fin_body,
        grid=(_G,),
        in_specs=[
            pl.BlockSpec((2, _R, 48), lambda i: (0, i, 0)),
            pl.BlockSpec((1, 2, _R, DEG_W), lambda i: (0, 0, i, 0)),
        ],
        out_specs=pl.BlockSpec((_R, 40), lambda i: (i, 0)),
        out_shape=jax.ShapeDtypeStruct((N_NODES, 40), f32),
    )(q, deg_parts[None])
    return out


# speedup vs baseline: 1.0012x; 1.0012x over previous
import functools

import jax
import jax.numpy as jnp
from jax import lax
from jax.experimental import pallas as pl
from jax.experimental.pallas import tpu as pltpu
from jax.experimental.pallas import tpu_sc as plsc

N_NODES = 10000
N_EDGES = 320000
NC = 2
NS = 16
NW = NC * NS
B_E = 64
K_CH = 158
K2 = 314
E_W = K_CH * B_E
N_ACC = 10112
RPT = N_ACC // NS
DEG_W = 8


def _sc_conv_call(table, row3, col3, zeros, stage_table=False):
    D = table.shape[1]
    mesh = plsc.VectorSubcoreMesh(
        core_axis_name="c", subcore_axis_name="s", num_cores=NC, num_subcores=NS
    )
    scratch = [
        pltpu.VMEM((K_CH, B_E), jnp.int32),
        pltpu.VMEM((K_CH, B_E), jnp.int32),
        pltpu.VMEM((B_E, D), jnp.float32),
        pltpu.VMEM((B_E, D), jnp.float32),
        pltpu.SemaphoreType.DMA,
        pltpu.SemaphoreType.DMA,
        pltpu.VMEM_SHARED((N_ACC, D), jnp.float32),
    ]
    if stage_table:
        scratch.append(pltpu.VMEM_SHARED((N_ACC, D), jnp.float32))

    @functools.partial(
        pl.kernel,
        out_type=jax.ShapeDtypeStruct((NC, N_ACC, D), jnp.float32),
        mesh=mesh,
        scratch_types=scratch,
        compiler_params=pltpu.CompilerParams(use_tc_tiling_on_sc=False),
    )
    def conv(table_h, row_h, col_h, zeros_h, out_h, rowv, colv, buf0, buf1,
             sem0, sem1, acc, *maybe_tab):
        c = lax.axis_index("c")
        s = lax.axis_index("s")
        w = c * NS + s
        pltpu.sync_copy(zeros_h.at[pl.ds(s * RPT, RPT)], acc.at[pl.ds(s * RPT, RPT)])
        if stage_table:
            table_src = maybe_tab[0]
            pltpu.sync_copy(
                table_h.at[pl.ds(s * RPT, RPT)], table_src.at[pl.ds(s * RPT, RPT)]
            )
        else:
            table_src = table_h
        pltpu.sync_copy(row_h.at[w], rowv)
        pltpu.sync_copy(col_h.at[w], colv)
        plsc.subcore_barrier()

        pltpu.async_copy(table_src.at[rowv.at[0]], buf0, sem0)

        def body(jj, carry):
            j0 = 2 * jj
            pltpu.async_copy(table_src.at[rowv.at[j0 + 1]], buf1, sem1)
            pltpu.make_async_copy(table_src.at[rowv.at[j0]], buf0, sem0).wait()
            pltpu.sync_copy(buf0, acc.at[colv.at[j0]], add=True)
            pltpu.async_copy(table_src.at[rowv.at[j0 + 2]], buf0, sem0)
            pltpu.make_async_copy(table_src.at[rowv.at[j0 + 1]], buf1, sem1).wait()
            pltpu.sync_copy(buf1, acc.at[colv.at[j0 + 1]], add=True)
            return carry

        lax.fori_loop(0, (K_CH - 2) // 2, body, 0)
        pltpu.async_copy(table_src.at[rowv.at[K_CH - 1]], buf1, sem1)
        pltpu.make_async_copy(table_src.at[rowv.at[K_CH - 2]], buf0, sem0).wait()
        pltpu.sync_copy(buf0, acc.at[colv.at[K_CH - 2]], add=True)
        pltpu.make_async_copy(table_src.at[rowv.at[K_CH - 1]], buf1, sem1).wait()
        pltpu.sync_copy(buf1, acc.at[colv.at[K_CH - 1]], add=True)

        plsc.subcore_barrier()
        pltpu.sync_copy(
            acc.at[pl.ds(s * RPT, RPT)], out_h.at[c, pl.ds(s * RPT, RPT)]
        )

    return conv(table, row3, col3, zeros)


def _sc_conv128_call(table, row2, col2, zeros):
    DH = 64
    mesh = plsc.VectorSubcoreMesh(
        core_axis_name="c", subcore_axis_name="s", num_cores=NC, num_subcores=NS
    )

    @functools.partial(
        pl.kernel,
        out_type=jax.ShapeDtypeStruct((2, N_ACC, DH), jnp.float32),
        mesh=mesh,
        scratch_types=[
            pltpu.VMEM((K2, B_E), jnp.int32),
            pltpu.VMEM((K2, B_E), jnp.int32),
            pltpu.VMEM((B_E, DH), jnp.float32),
            pltpu.VMEM((B_E, DH), jnp.float32),
            pltpu.SemaphoreType.DMA,
            pltpu.SemaphoreType.DMA,
            pltpu.VMEM_SHARED((N_ACC, DH), jnp.float32),
            pltpu.VMEM_SHARED((N_ACC, DH), jnp.float32),
        ],
        compiler_params=pltpu.CompilerParams(use_tc_tiling_on_sc=False),
    )
    def conv(table_h, row_h, col_h, zeros_h, out_h, rowv, colv, buf0, buf1,
             sem0, sem1, acc, tab):
        c = lax.axis_index("c")
        s = lax.axis_index("s")
        pltpu.sync_copy(row_h.at[s], rowv)
        pltpu.sync_copy(col_h.at[s], colv)
        pltpu.sync_copy(zeros_h.at[pl.ds(s * RPT, RPT)], acc.at[pl.ds(s * RPT, RPT)])
        pltpu.sync_copy(
            table_h.at[pl.ds(s * RPT, RPT), pl.ds(c * DH, DH)],
            tab.at[pl.ds(s * RPT, RPT)],
        )
        plsc.subcore_barrier()

        pltpu.async_copy(tab.at[rowv.at[0]], buf0, sem0)

        def body(jj, carry):
            j0 = 2 * jj
            pltpu.async_copy(tab.at[rowv.at[j0 + 1]], buf1, sem1)
            pltpu.make_async_copy(tab.at[rowv.at[j0]], buf0, sem0).wait()
            pltpu.sync_copy(buf0, acc.at[colv.at[j0]], add=True)
            pltpu.async_copy(tab.at[rowv.at[j0 + 2]], buf0, sem0)
            pltpu.make_async_copy(tab.at[rowv.at[j0 + 1]], buf1, sem1).wait()
            pltpu.sync_copy(buf1, acc.at[colv.at[j0 + 1]], add=True)
            return carry

        lax.fori_loop(0, (K2 - 2) // 2, body, 0)
        pltpu.async_copy(tab.at[rowv.at[K2 - 1]], buf1, sem1)
        pltpu.make_async_copy(tab.at[rowv.at[K2 - 2]], buf0, sem0).wait()
        pltpu.sync_copy(buf0, acc.at[colv.at[K2 - 2]], add=True)
        pltpu.make_async_copy(tab.at[rowv.at[K2 - 1]], buf1, sem1).wait()
        pltpu.sync_copy(buf1, acc.at[colv.at[K2 - 1]], add=True)

        plsc.subcore_barrier()
        pltpu.sync_copy(
            acc.at[pl.ds(s * RPT, RPT)], out_h.at[c, pl.ds(s * RPT, RPT)]
        )

    return conv(table, row2, col2, zeros)


B_D = 128
K_D = K_CH // 2


def _sc_deg_call(ones_blk, col3d, zeros):
    mesh = plsc.VectorSubcoreMesh(
        core_axis_name="c", subcore_axis_name="s", num_cores=NC, num_subcores=NS
    )

    @functools.partial(
        pl.kernel,
        out_type=jax.ShapeDtypeStruct((NC, N_ACC, DEG_W), jnp.float32),
        mesh=mesh,
        scratch_types=[
            pltpu.VMEM((K_D, B_D), jnp.int32),
            pltpu.VMEM((B_D, DEG_W), jnp.float32),
            pltpu.VMEM_SHARED((N_ACC, DEG_W), jnp.float32),
        ],
        compiler_params=pltpu.CompilerParams(use_tc_tiling_on_sc=False),
    )
    def degk(ones_h, col_h, zeros_h, out_h, colv, onesv, acc):
        c = lax.axis_index("c")
        s = lax.axis_index("s")
        w = c * NS + s
        pltpu.sync_copy(zeros_h.at[pl.ds(s * RPT, RPT)], acc.at[pl.ds(s * RPT, RPT)])
        pltpu.sync_copy(col_h.at[w], colv)
        pltpu.sync_copy(ones_h, onesv)
        plsc.subcore_barrier()

        def body(j, carry):
            pltpu.sync_copy(onesv, acc.at[colv.at[j]], add=True)
            return carry

        lax.fori_loop(0, K_D, body, 0)
        plsc.subcore_barrier()
        pltpu.sync_copy(
            acc.at[pl.ds(s * RPT, RPT)], out_h.at[c, pl.ds(s * RPT, RPT)]
        )

    return degk(ones_blk, col3d, zeros)


def _dis_from(d_ref):
    deg = d_ref[0, :, :, 0:1].sum(axis=0)
    return jnp.where(deg > 0.0, lax.rsqrt(deg), 0.0)


def _mm0_body(x_ref, w_ref, o_ref):
    o_ref[...] = jnp.dot(x_ref[...], w_ref[...], preferred_element_type=jnp.float32)


def _scale_body(h_ref, d_ref, o_ref):
    o_ref[...] = h_ref[...] * _dis_from(d_ref)


def _mm2_body(pa_ref, pb_ref, d_ref, wa_ref, wb_ref, o_ref):
    dis = _dis_from(d_ref)
    out1a = pa_ref[0] * dis
    out1b = pb_ref[0] * dis
    h2 = jnp.dot(out1a, wa_ref[...], preferred_element_type=jnp.float32)
    h2 += jnp.dot(out1b, wb_ref[...], preferred_element_type=jnp.float32)
    o_ref[...] = h2 * dis


def _fin_body(q_ref, d_ref, o_ref):
    dis = _dis_from(d_ref)
    a = ((q_ref[0] + q_ref[1]) * dis)[:, :40]
    m = jnp.max(a, axis=1, keepdims=True)
    e = jnp.exp(a - m)
    lse = jnp.log(jnp.sum(e, axis=1, keepdims=True))
    o_ref[...] = a - m - lse


_R = 632
_G = N_ACC // _R


@jax.jit
def kernel(x, edge_index, W1, W2):
    f32 = jnp.float32
    row = edge_index[0]
    col = edge_index[1]
    pad = NW * E_W - N_EDGES
    row_p = jnp.concatenate([row, jnp.zeros((pad,), jnp.int32)])
    col_p = jnp.concatenate([col, jnp.full((pad,), N_NODES, jnp.int32)])
    row3 = row_p.reshape(NW, K_CH, B_E)
    col3 = col_p.reshape(NW, K_CH, B_E)
    pad2 = NS * K2 * B_E - N_EDGES
    row2 = jnp.concatenate([row, jnp.zeros((pad2,), jnp.int32)]).reshape(NS, K2, B_E)
    col2 = jnp.concatenate([col, jnp.full((pad2,), N_NODES, jnp.int32)]).reshape(
        NS, K2, B_E
    )

    zeros64 = jnp.zeros((N_ACC, 64), f32)
    zeros48 = jnp.zeros((N_ACC, 48), f32)
    zeros16 = jnp.zeros((N_ACC, DEG_W), f32)
    ones_blk = jnp.ones((B_D, DEG_W), f32)

    h1 = pl.pallas_call(
        _mm0_body,
        grid=(_G,),
        in_specs=[
            pl.BlockSpec((_R, 128), lambda i: (i, 0)),
            pl.BlockSpec((128, 128), lambda i: (0, 0)),
        ],
        out_specs=pl.BlockSpec((_R, 128), lambda i: (i, 0)),
        out_shape=jax.ShapeDtypeStruct((N_NODES, 128), f32),
    )(x, W1)
    deg_parts = _sc_deg_call(
        ones_blk, col3.reshape(NW, K_D, B_D), zeros16
    )

    h1p = pl.pallas_call(
        _scale_body,
        grid=(_G,),
        in_specs=[
            pl.BlockSpec((_R, 128), lambda i: (i, 0)),
            pl.BlockSpec((1, 2, _R, DEG_W), lambda i: (0, 0, i, 0)),
        ],
        out_specs=pl.BlockSpec((_R, 128), lambda i: (i, 0)),
        out_shape=jax.ShapeDtypeStruct((N_ACC, 128), f32),
    )(h1, deg_parts[None])

    phalves = _sc_conv128_call(h1p, row2, col2, zeros64)

    W2p = jnp.pad(W2, ((0, 0), (0, 8)))
    h2p = pl.pallas_call(
        _mm2_body,
        grid=(_G,),
        in_specs=[
            pl.BlockSpec((1, _R, 64), lambda i: (0, i, 0)),
            pl.BlockSpec((1, _R, 64), lambda i: (1, i, 0)),
            pl.BlockSpec((1, 2, _R, DEG_W), lambda i: (0, 0, i, 0)),
            pl.BlockSpec((64, 48), lambda i: (0, 0)),
            pl.BlockSpec((64, 48), lambda i: (0, 0)),
        ],
        out_specs=pl.BlockSpec((_R, 48), lambda i: (i, 0)),
        out_shape=jax.ShapeDtypeStruct((N_ACC, 48), f32),
    )(phalves, phalves, deg_parts[None], W2p[:64], W2p[64:])

    q = _sc_conv_call(h2p, row3, col3, zeros48, stage_table=True)

    out = pl.pallas_call(
        _fin_body,
        grid=(_G,),
        in_specs=[
            pl.BlockSpec((2, _R, 48), lambda i: (0, i, 0)),
            pl.BlockSpec((1, 2, _R, DEG_W), lambda i: (0, 0, i, 0)),
        ],
        out_specs=pl.BlockSpec((_R, 40), lambda i: (i, 0)),
        out_shape=jax.ShapeDtypeStruct((N_NODES, 40), f32),
    )(q, deg_parts[None])
    return out
